# baseline (speedup 1.0000x reference)
import functools

import numpy as np
import jax
import jax.numpy as jnp
from jax import lax
from jax.experimental import pallas as pl
from jax.experimental.pallas import tpu as pltpu
from jax.experimental.pallas import tpu_sc as plsc

F = 26
D = 16
B = 16384
N = B * F
NC, NS = 2, 16
NW = NC * NS
PER_W = N // NW
IR = PER_W // 128
KP = 4
CHUNK = KP * 128
NCHUNK = PER_W // CHUNK

H0, H1, H2, H3 = 128, 64, 32, 1
BLK = 2048

_OFFS = np.arange(F, dtype=np.int32) * 100000

E1 = 2600000
E1W = 81248


@functools.lru_cache(maxsize=None)
def _make_sc_flatten():
    mesh = plsc.VectorSubcoreMesh(core_axis_name="c", subcore_axis_name="s")

    @functools.partial(
        pl.kernel,
        out_type=jax.ShapeDtypeStruct((E1, 1), jnp.float32),
        mesh=mesh,
        scratch_types=[],
        compiler_params=pltpu.CompilerParams(use_tc_tiling_on_sc=False),
    )
    def _sc_flatten(src, dst):
        wid = lax.axis_index("s") * NC + lax.axis_index("c")
        base = wid * E1W
        pltpu.sync_copy(src.at[pl.ds(base, E1W)], dst.at[pl.ds(base, E1W)])

        @pl.when(wid == NW - 1)
        def _():
            pltpu.sync_copy(src.at[pl.ds(NW * E1W, 64)],
                            dst.at[pl.ds(NW * E1W, 64)])

    return _sc_flatten


@functools.lru_cache(maxsize=None)
def _make_sc_gather():
    mesh = plsc.VectorSubcoreMesh(core_axis_name="c", subcore_axis_name="s")

    @functools.partial(
        pl.kernel,
        out_type=(
            jax.ShapeDtypeStruct((N, D), jnp.float32),
            jax.ShapeDtypeStruct((N,), jnp.float32),
        ),
        mesh=mesh,
        scratch_types=[
            pltpu.VMEM((IR, 128), jnp.int32),
            pltpu.VMEM((CHUNK, D), jnp.float32),
            pltpu.VMEM((CHUNK, D), jnp.float32),
            pltpu.VMEM((PER_W,), jnp.float32),
            pltpu.SemaphoreType.DMA,
            pltpu.SemaphoreType.DMA,
        ],
        compiler_params=pltpu.CompilerParams(use_tc_tiling_on_sc=False),
    )
    def _sc_gather(idx_hbm2, emb2_hbm, emb1_hbm, v_out, w_out,
                   idx_v, rows_a, rows_b, vals_v, sem2, sem1):
        wid = lax.axis_index("s") * NC + lax.axis_index("c")
        base_w = wid * PER_W

        pltpu.sync_copy(idx_hbm2.at[pl.ds(wid * IR, IR)], idx_v)

        def fire1(i, carry):
            for j in range(8):
                pltpu.async_copy(
                    emb1_hbm.at[idx_v.at[i * 8 + j]],
                    vals_v.at[pl.ds((i * 8 + j) * 128, 128)], sem1)
            return carry

        lax.fori_loop(0, IR // 8, fire1, 0)

        def fire2(r, buf):
            for j in range(KP):
                pltpu.async_copy(
                    emb2_hbm.at[idx_v.at[r * KP + j]],
                    buf.at[pl.ds(j * 128, 128)], sem2)

        def drain2(r, buf):
            for j in range(KP):
                pltpu.make_async_copy(
                    emb2_hbm.at[idx_v.at[r * KP + j]],
                    buf.at[pl.ds(j * 128, 128)], sem2).wait()

        fire2(0, rows_a)

        def round_pair(p, carry):
            r = p * 2
            fire2(r + 1, rows_b)
            drain2(r, rows_a)
            pltpu.sync_copy(rows_a, v_out.at[pl.ds(base_w + r * CHUNK, CHUNK)])

            @pl.when(p < NCHUNK // 2 - 1)
            def _():
                fire2(r + 2, rows_a)

            drain2(r + 1, rows_b)
            pltpu.sync_copy(rows_b, v_out.at[pl.ds(base_w + (r + 1) * CHUNK, CHUNK)])
            return carry

        lax.fori_loop(0, NCHUNK // 2, round_pair, 0)

        def drain1(i, carry):
            for j in range(8):
                pltpu.make_async_copy(
                    emb1_hbm.at[idx_v.at[i * 8 + j]],
                    vals_v.at[pl.ds((i * 8 + j) * 128, 128)], sem1).wait()
            return carry

        lax.fori_loop(0, IR // 8, drain1, 0)
        pltpu.sync_copy(vals_v, w_out.at[pl.ds(base_w, PER_W)])

    return _sc_gather


def _mlp_body(w_ref, v_ref, w0a, w0b, b0, w1, b1, w2, b2, w3, b3, o_ref):
    h = jnp.dot(w_ref[...], w0a[...], preferred_element_type=jnp.float32)
    h = h + jnp.dot(v_ref[...], w0b[...], preferred_element_type=jnp.float32)
    h = jnp.maximum(h + b0[...], 0.0)
    h = jnp.maximum(jnp.dot(h, w1[...], preferred_element_type=jnp.float32) + b1[...], 0.0)
    h = jnp.maximum(jnp.dot(h, w2[...], preferred_element_type=jnp.float32) + b2[...], 0.0)
    z = jnp.dot(h, w3[...], preferred_element_type=jnp.float32) + b3[...]
    o_ref[...] = 1.0 / (1.0 + jnp.exp(-z))


def _full(shape):
    return pl.BlockSpec(shape, lambda i: (0, 0))


def _mlp(w, v, W0a, W0b, b0, W1, b1, W2, b2, W3, b3):
    return pl.pallas_call(
        _mlp_body,
        grid=(B // BLK,),
        in_specs=[
            pl.BlockSpec((BLK, F), lambda i: (i, 0)),
            pl.BlockSpec((BLK, F * D), lambda i: (i, 0)),
            _full((F, H0)), _full((F * D, H0)), _full((1, H0)),
            _full((H0, H1)), _full((1, H1)),
            _full((H1, H2)), _full((1, H2)),
            _full((H2, H3)), _full((1, H3)),
        ],
        out_specs=pl.BlockSpec((BLK, 1), lambda i: (i, 0)),
        out_shape=jax.ShapeDtypeStruct((B, H3), jnp.float32),
    )(w, v, W0a, W0b, b0, W1, b1, W2, b2, W3, b3)


def kernel(x, emb1, emb2, Wm0, bm0, Wm1, bm1, Wm2, bm2, Wm3, bm3):
    idx = (x + jnp.asarray(_OFFS)[None, :]).reshape(-1)
    emb1_lin = _make_sc_flatten()(emb1)
    v_flat, w_flat = _make_sc_gather()(
        idx.reshape(N // 128, 128), emb2, emb1_lin.reshape(-1))
    w = w_flat.reshape(B, F)
    v = v_flat.reshape(B, F * D)
    return _mlp(
        w, v, Wm0[:F], Wm0[F:], bm0.reshape(1, -1),
        Wm1, bm1.reshape(1, -1), Wm2, bm2.reshape(1, -1),
        Wm3, bm3.reshape(1, -1),
    )


# speedup vs baseline: 6.3974x; 6.3974x over previous
import functools

import numpy as np
import jax
import jax.numpy as jnp
from jax import lax
from jax.experimental import pallas as pl
from jax.experimental.pallas import tpu as pltpu
from jax.experimental.pallas import tpu_sc as plsc

F = 26
D = 16
B = 16384
N = B * F
NC, NS = 2, 16
NW = NC * NS
PER_W = N // NW
IR = PER_W // 128
KP = 4
CHUNK = KP * 128
NCHUNK = PER_W // CHUNK

H0, H1, H2, H3 = 128, 64, 32, 1
BLK = 2048

_OFFS = np.arange(F, dtype=np.int32) * 100000

@functools.lru_cache(maxsize=None)
def _make_sc_gather():
    mesh = plsc.VectorSubcoreMesh(core_axis_name="c", subcore_axis_name="s")

    @functools.partial(
        pl.kernel,
        out_type=(
            jax.ShapeDtypeStruct((N, D), jnp.float32),
            jax.ShapeDtypeStruct((N,), jnp.float32),
        ),
        mesh=mesh,
        scratch_types=[
            pltpu.VMEM((IR, 128), jnp.int32),
            pltpu.VMEM((CHUNK, D), jnp.float32),
            pltpu.VMEM((CHUNK, D), jnp.float32),
            pltpu.VMEM((PER_W,), jnp.float32),
            pltpu.SemaphoreType.DMA,
            pltpu.SemaphoreType.DMA,
        ],
        compiler_params=pltpu.CompilerParams(use_tc_tiling_on_sc=False),
    )
    def _sc_gather(idx_hbm2, emb2_hbm, emb1_hbm, v_out, w_out,
                   idx_v, rows_a, rows_b, vals_v, sem2, sem1):
        wid = lax.axis_index("s") * NC + lax.axis_index("c")
        base_w = wid * PER_W

        pltpu.sync_copy(idx_hbm2.at[pl.ds(wid * IR, IR)], idx_v)

        def fire1(i, carry):
            for j in range(8):
                pltpu.async_copy(
                    emb1_hbm.at[idx_v.at[i * 8 + j]],
                    vals_v.at[pl.ds((i * 8 + j) * 128, 128)], sem1)
            return carry

        lax.fori_loop(0, IR // 8, fire1, 0)

        def fire2(r, buf):
            for j in range(KP):
                pltpu.async_copy(
                    emb2_hbm.at[idx_v.at[r * KP + j]],
                    buf.at[pl.ds(j * 128, 128)], sem2)

        def drain2(r, buf):
            for j in range(KP):
                pltpu.make_async_copy(
                    emb2_hbm.at[idx_v.at[r * KP + j]],
                    buf.at[pl.ds(j * 128, 128)], sem2).wait()

        fire2(0, rows_a)

        def round_pair(p, carry):
            r = p * 2
            fire2(r + 1, rows_b)
            drain2(r, rows_a)
            pltpu.sync_copy(rows_a, v_out.at[pl.ds(base_w + r * CHUNK, CHUNK)])

            @pl.when(p < NCHUNK // 2 - 1)
            def _():
                fire2(r + 2, rows_a)

            drain2(r + 1, rows_b)
            pltpu.sync_copy(rows_b, v_out.at[pl.ds(base_w + (r + 1) * CHUNK, CHUNK)])
            return carry

        lax.fori_loop(0, NCHUNK // 2, round_pair, 0)

        def drain1(i, carry):
            for j in range(8):
                pltpu.make_async_copy(
                    emb1_hbm.at[idx_v.at[i * 8 + j]],
                    vals_v.at[pl.ds((i * 8 + j) * 128, 128)], sem1).wait()
            return carry

        lax.fori_loop(0, IR // 8, drain1, 0)
        pltpu.sync_copy(vals_v, w_out.at[pl.ds(base_w, PER_W)])

    return _sc_gather


def _mlp_body(w_ref, v_ref, w0a, w0b, b0, w1, b1, w2, b2, w3, b3, o_ref):
    h = jnp.dot(w_ref[...], w0a[...], preferred_element_type=jnp.float32)
    h = h + jnp.dot(v_ref[...], w0b[...], preferred_element_type=jnp.float32)
    h = jnp.maximum(h + b0[...], 0.0)
    h = jnp.maximum(jnp.dot(h, w1[...], preferred_element_type=jnp.float32) + b1[...], 0.0)
    h = jnp.maximum(jnp.dot(h, w2[...], preferred_element_type=jnp.float32) + b2[...], 0.0)
    z = jnp.dot(h, w3[...], preferred_element_type=jnp.float32) + b3[...]
    o_ref[...] = 1.0 / (1.0 + jnp.exp(-z))


def _full(shape):
    return pl.BlockSpec(shape, lambda i: (0, 0))


def _mlp(w, v, W0a, W0b, b0, W1, b1, W2, b2, W3, b3):
    return pl.pallas_call(
        _mlp_body,
        grid=(B // BLK,),
        in_specs=[
            pl.BlockSpec((BLK, F), lambda i: (i, 0)),
            pl.BlockSpec((BLK, F * D), lambda i: (i, 0)),
            _full((F, H0)), _full((F * D, H0)), _full((1, H0)),
            _full((H0, H1)), _full((1, H1)),
            _full((H1, H2)), _full((1, H2)),
            _full((H2, H3)), _full((1, H3)),
        ],
        out_specs=pl.BlockSpec((BLK, 1), lambda i: (i, 0)),
        out_shape=jax.ShapeDtypeStruct((B, H3), jnp.float32),
    )(w, v, W0a, W0b, b0, W1, b1, W2, b2, W3, b3)


def kernel(x, emb1, emb2, Wm0, bm0, Wm1, bm1, Wm2, bm2, Wm3, bm3):
    idx = (x + jnp.asarray(_OFFS)[None, :]).reshape(-1)
    v_flat, w_flat = _make_sc_gather()(
        idx.reshape(N // 128, 128), emb2, emb1.reshape(-1))
    w = w_flat.reshape(B, F)
    v = v_flat.reshape(B, F * D)
    return _mlp(
        w, v, Wm0[:F], Wm0[F:], bm0.reshape(1, -1),
        Wm1, bm1.reshape(1, -1), Wm2, bm2.reshape(1, -1),
        Wm3, bm3.reshape(1, -1),
    )
